# baseline (speedup 1.0000x reference)
import functools

import jax
import jax.numpy as jnp
from jax import lax
from jax.experimental import pallas as pl
from jax.experimental.pallas import tpu as pltpu
from jax.experimental.pallas import tpu_sc as plsc

NC = 2
NS = 16
NW = NC * NS
L = 16


def _mf_body(chunk, k_dim, uid_hbm, iid_hbm, bias_hbm, utab_hbm, itab_hbm,
             wv_hbm, bv_hbm, out_hbm,
             uidx_v, iidx_v, urows_v, irows_v, bf_v, out_v, wv_v, bv_v,
             sem_u, sem_i):
    wid = lax.axis_index("s") * NC + lax.axis_index("c")
    base = wid * chunk

    pltpu.sync_copy(uid_hbm.at[pl.ds(base, chunk)], uidx_v)
    pltpu.sync_copy(iid_hbm.at[pl.ds(base, chunk)], iidx_v)
    cu = pltpu.async_copy(utab_hbm.at[uidx_v], urows_v, sem_u)
    ci = pltpu.async_copy(itab_hbm.at[iidx_v], irows_v, sem_i)
    pltpu.sync_copy(bias_hbm.at[pl.ds(base, chunk)], bf_v)
    pltpu.sync_copy(wv_hbm, wv_v)
    pltpu.sync_copy(bv_hbm, bv_v)
    cu.wait()
    ci.wait()

    wvec = wv_v[...]
    bvec = bv_v[...]
    lane = lax.iota(jnp.int32, L)
    rot_idx = {h: jnp.bitwise_xor(lane, h) for h in (8, 4, 2, 1)}
    keep_lo = {h: (jnp.bitwise_and(lane, h) == 0) for h in (8, 4, 2, 1)}
    brev = (((lane & 1) << 3) | ((lane & 2) << 1)
            | ((lane & 4) >> 1) | ((lane & 8) >> 3))

    def perm(x, idx):
        return x.at[idx].get(mode="promise_in_bounds")

    def group(g, carry):
        base = g * L
        vecs = []
        for j in range(L):
            r = base + j
            s = jnp.zeros((L,), jnp.float32)
            for half in range(k_dim // L):
                u = urows_v[r, pl.ds(half * L, L)]
                it = irows_v[r, pl.ds(half * L, L)]
                s = s + u * it
            vecs.append(s)
        for h in (8, 4, 2, 1):
            nxt = []
            for n in range(len(vecs) // 2):
                x, y = vecs[2 * n], vecs[2 * n + 1]
                x2 = x + perm(x, rot_idx[h])
                y2 = y + perm(y, rot_idx[h])
                nxt.append(jnp.where(keep_lo[h], x2, y2))
            vecs = nxt
        dots = perm(vecs[0], brev)
        acc = dots + bf_v[pl.ds(base, L)] * wvec + bvec
        out_v[pl.ds(base, L)] = acc
        return carry

    lax.fori_loop(0, chunk // L, group, 0)
    pltpu.sync_copy(out_v, out_hbm.at[pl.ds(base, chunk)])


@functools.partial(jax.jit, static_argnums=())
def kernel(user_ids, item_ids, bias_feat, user_table, item_table, dense_W,
           dense_b):
    batch = user_ids.shape[0]
    k_dim = user_table.shape[1]
    chunk = batch // NW

    mesh = plsc.VectorSubcoreMesh(core_axis_name="c", subcore_axis_name="s")
    mf = pl.kernel(
        functools.partial(_mf_body, chunk, k_dim),
        out_type=jax.ShapeDtypeStruct((batch,), jnp.float32),
        mesh=mesh,
        scratch_types=[
            pltpu.VMEM((chunk,), jnp.int32),
            pltpu.VMEM((chunk,), jnp.int32),
            pltpu.VMEM((chunk, k_dim), jnp.float32),
            pltpu.VMEM((chunk, k_dim), jnp.float32),
            pltpu.VMEM((chunk,), jnp.float32),
            pltpu.VMEM((chunk,), jnp.float32),
            pltpu.VMEM((L,), jnp.float32),
            pltpu.VMEM((L,), jnp.float32),
            pltpu.SemaphoreType.DMA,
            pltpu.SemaphoreType.DMA,
        ],
        compiler_params=pltpu.CompilerParams(use_tc_tiling_on_sc=False),
    )

    wv = jnp.broadcast_to(dense_W.reshape(()), (L,)).astype(jnp.float32)
    bv = jnp.broadcast_to(dense_b.reshape(()), (L,)).astype(jnp.float32)
    out = mf(user_ids.reshape(batch), item_ids.reshape(batch),
             bias_feat.reshape(batch), user_table, item_table, wv, bv)
    return out.reshape(batch, 1)


# speedup vs baseline: 2.2994x; 2.2994x over previous
import functools

import jax
import jax.numpy as jnp
from jax import lax
from jax.experimental import pallas as pl
from jax.experimental.pallas import tpu as pltpu
from jax.experimental.pallas import tpu_sc as plsc

NC = 2
NS = 16
NW = NC * NS
L = 16


def _mf_body(chunk, k_dim, uid_hbm, iid_hbm, bias_hbm, utab_hbm, itab_hbm,
             wv_hbm, bv_hbm, out_hbm,
             uidx_v, iidx_v, ublk_v, iblk_v, bf_v, out_v, wv_v, bv_v,
             sem_u, sem_i):
    wid = lax.axis_index("s") * NC + lax.axis_index("c")
    base = wid * chunk
    n_win = chunk // L

    pltpu.sync_copy(uid_hbm.at[pl.ds(base, chunk)], uidx_v)
    pltpu.sync_copy(iid_hbm.at[pl.ds(base, chunk)], iidx_v)
    pltpu.sync_copy(bias_hbm.at[pl.ds(base, chunk)], bf_v)
    pltpu.sync_copy(wv_hbm, wv_v)
    pltpu.sync_copy(bv_hbm, bv_v)

    def fire(w):
        p = jnp.bitwise_and(w, 1)
        uvec = uidx_v[pl.ds(w * L, L)] >> 3
        ivec = iidx_v[pl.ds(w * L, L)] >> 3
        for l in range(L):
            pltpu.async_copy(utab_hbm.at[uvec[l]], ublk_v.at[p, l],
                             sem_u.at[p])
            pltpu.async_copy(itab_hbm.at[ivec[l]], iblk_v.at[p, l],
                             sem_i.at[p])

    wvec = wv_v[...]
    bvec = bv_v[...]
    lane = lax.iota(jnp.int32, L)
    keep_lo = {h: (jnp.bitwise_and(lane, h) == 0) for h in (8, 4, 2, 1)}
    brev = (((lane & 1) << 3) | ((lane & 2) << 1)
            | ((lane & 4) >> 1) | ((lane & 8) >> 3))

    def perm(x, idx):
        return x.at[idx].get(mode="promise_in_bounds")

    fire(0)

    def window(w, carry):
        @pl.when(w + 1 < n_win)
        def _():
            fire(w + 1)
        p = jnp.bitwise_and(w, 1)
        pltpu.make_async_copy(utab_hbm.at[pl.ds(0, L)], ublk_v.at[p],
                              sem_u.at[p]).wait()
        pltpu.make_async_copy(itab_hbm.at[pl.ds(0, L)], iblk_v.at[p],
                              sem_i.at[p]).wait()
        usub = jnp.bitwise_and(uidx_v[pl.ds(w * L, L)], 7)
        isub = jnp.bitwise_and(iidx_v[pl.ds(w * L, L)], 7)
        vecs = []
        for l in range(L):
            su = usub[l]
            si = isub[l]
            s = jnp.zeros((L,), jnp.float32)
            for half in range(k_dim // L):
                u = ublk_v[p, l, su, pl.ds(half * L, L)]
                it = iblk_v[p, l, si, pl.ds(half * L, L)]
                s = s + u * it
            vecs.append(s)
        for h in (8, 4, 2, 1):
            nxt = []
            for n in range(len(vecs) // 2):
                x, y = vecs[2 * n], vecs[2 * n + 1]
                x2 = x + perm(x, jnp.bitwise_xor(lane, h))
                y2 = y + perm(y, jnp.bitwise_xor(lane, h))
                nxt.append(jnp.where(keep_lo[h], x2, y2))
            vecs = nxt
        dots = perm(vecs[0], brev)
        acc = dots + bf_v[pl.ds(w * L, L)] * wvec + bvec
        out_v[pl.ds(w * L, L)] = acc
        return carry

    lax.fori_loop(0, n_win, window, 0)
    pltpu.sync_copy(out_v, out_hbm.at[pl.ds(base, chunk)])


@jax.jit
def kernel(user_ids, item_ids, bias_feat, user_table, item_table, dense_W,
           dense_b):
    batch = user_ids.shape[0]
    k_dim = user_table.shape[1]
    chunk = batch // NW

    utab3 = user_table.reshape(user_table.shape[0] // 8, 8, k_dim)
    itab3 = item_table.reshape(item_table.shape[0] // 8, 8, k_dim)

    mesh = plsc.VectorSubcoreMesh(core_axis_name="c", subcore_axis_name="s")
    mf = pl.kernel(
        functools.partial(_mf_body, chunk, k_dim),
        out_type=jax.ShapeDtypeStruct((batch,), jnp.float32),
        mesh=mesh,
        scratch_types=[
            pltpu.VMEM((chunk,), jnp.int32),
            pltpu.VMEM((chunk,), jnp.int32),
            pltpu.VMEM((2, L, 8, k_dim), jnp.float32),
            pltpu.VMEM((2, L, 8, k_dim), jnp.float32),
            pltpu.VMEM((chunk,), jnp.float32),
            pltpu.VMEM((chunk,), jnp.float32),
            pltpu.VMEM((L,), jnp.float32),
            pltpu.VMEM((L,), jnp.float32),
            pltpu.SemaphoreType.DMA((2,)),
            pltpu.SemaphoreType.DMA((2,)),
        ],
    )

    wv = jnp.broadcast_to(dense_W.reshape(()), (L,)).astype(jnp.float32)
    bv = jnp.broadcast_to(dense_b.reshape(()), (L,)).astype(jnp.float32)
    out = mf(user_ids.reshape(batch), item_ids.reshape(batch),
             bias_feat.reshape(batch), utab3, itab3, wv, bv)
    return out.reshape(batch, 1)
